# Initial kernel scaffold; baseline (speedup 1.0000x reference)
import jax
import jax.numpy as jnp
from jax.experimental import pallas as pl


def kernel(x, params, edge_index):
    raise NotImplementedError("write your pallas kernel here")


# speedup vs baseline: 15.6890x; 15.6890x over previous
import functools

import jax
import jax.numpy as jnp
from jax import lax
from jax.experimental import pallas as pl
from jax.experimental.pallas import tpu as pltpu
from jax.experimental.pallas import tpu_sc as plsc

_HEX_OFFSETS = ((1, 0), (-1, 0), (0, 1), (0, -1), (1, -1), (-1, 1))
_NC, _NS = 2, 16
_NW = _NC * _NS
_G = 96

_SQRT_HALF = 0.7071067811865476


def _gelu(v):
    return v * 0.5 * (1.0 + lax.erf(v * _SQRT_HALF))


def _ln(v, g, b):
    m = jnp.mean(v, axis=-1, keepdims=True)
    c = v - m
    var = jnp.mean(c * c, axis=-1, keepdims=True)
    return c * lax.rsqrt(var + 1e-5) * g + b


def _mm(a, w):
    return lax.dot_general(a, w, (((1,), (0,)), ((), ())),
                           preferred_element_type=jnp.float32)


def _embed_body(B, xt_ref, wi_ref, bi_ref, g_ref, b_ref, out_ref):
    bi = pl.program_id(0)

    @pl.when(bi == B)
    def _pad():
        out_ref[...] = jnp.zeros_like(out_ref)

    @pl.when(bi < B)
    def _emb():
        h0 = _mm(xt_ref[0], wi_ref[...]) + bi_ref[...]
        out_ref[...] = _ln(_gelu(h0), g_ref[...], b_ref[...])


def _layer_body(h_ref, hs_ref, wm_ref, wu1_ref, wu2_ref, bu_ref, g_ref,
                b_ref, invdeg_ref, out_ref):
    h = h_ref[...]
    agg = _mm(hs_ref[...], wm_ref[...]) * invdeg_ref[...]
    o = _mm(h, wu1_ref[...]) + _mm(agg, wu2_ref[...]) + bu_ref[...]
    out_ref[...] = _ln(_gelu(o) + h, g_ref[...], b_ref[...])


def _heads_body(h_ref, wq_ref, bq_ref, wp_ref, bp_ref, w1_ref, b1_ref,
                w2_ref, b2_ref, wt_ref, bt_ref, p_ref, v_ref, t_ref):
    h = h_ref[...]
    p_ref[...] = (_mm(h, wp_ref[...]) + bp_ref[...])[None]
    s = _mm(h, wq_ref[...]) + bq_ref[...]
    e = jnp.exp(s - jnp.max(s))
    a = e / jnp.sum(e)
    gf = jnp.sum(h * a, axis=0, keepdims=True)
    v1 = _gelu(_mm(gf, w1_ref[...]) + b1_ref[...])
    v_ref[...] = jnp.tanh(_mm(v1, w2_ref[...]) + b2_ref[...])[None]
    t_ref[...] = (_mm(gf, wt_ref[...]) + bt_ref[...])[None]


@functools.partial(jax.jit, static_argnums=(2, 3))
def _sc_neighbor_sum(h_tab, nbr, num_chunks, H):
    mesh = plsc.VectorSubcoreMesh(core_axis_name="c", subcore_axis_name="s")
    nseg = H // 16
    outer = (num_chunks + _NW - 1) // _NW

    @functools.partial(
        pl.kernel, mesh=mesh,
        out_type=jax.ShapeDtypeStruct((num_chunks, _G, H), jnp.float32),
        scratch_types=[pltpu.VMEM((_G,), jnp.int32) for _ in range(6)]
        + [pltpu.VMEM((_G, H), jnp.float32) for _ in range(6)]
        + [pltpu.SemaphoreType.DMA],
    )
    def k(h_hbm, nbr_hbm, out_hbm, i0, i1, i2, i3, i4, i5,
          r0, r1, r2, r3, r4, r5, sem):
        idxs = (i0, i1, i2, i3, i4, i5)
        rows = (r0, r1, r2, r3, r4, r5)
        wid = lax.axis_index("s") * _NC + lax.axis_index("c")

        def chunk_body(ci, carry):
            c = wid + ci * _NW

            @pl.when(c < num_chunks)
            def _work():
                for o in range(6):
                    pltpu.sync_copy(nbr_hbm.at[o, c], idxs[o])
                cps = [pltpu.async_copy(h_hbm.at[idxs[o]], rows[o], sem)
                       for o in range(6)]
                for cp in cps:
                    cp.wait()

                def row_body(rr, rcarry):
                    for ss in range(nseg):
                        sl = pl.ds(ss * 16, 16)
                        r0[rr, sl] = ((r0[rr, sl] + r1[rr, sl])
                                      + (r2[rr, sl] + r3[rr, sl])
                                      + (r4[rr, sl] + r5[rr, sl]))
                    return rcarry

                lax.fori_loop(0, _G, row_body, 0)
                pltpu.sync_copy(r0, out_hbm.at[c])
            return carry

        lax.fori_loop(0, outer, chunk_body, 0)

    return k(h_tab, nbr)


def kernel(x, params, edge_index):
    B, C, N, _ = x.shape
    N2 = N * N
    H = params['Wi'].shape[1]
    total = B * N2
    zero_row = total
    tab_rows = (B + 1) * N2
    num_chunks = (total + _G - 1) // _G
    pad = num_chunks * _G - total

    f32 = jnp.float32
    xt = jnp.transpose(x.reshape(B, C, N2), (0, 2, 1))

    n = jnp.arange(N2, dtype=jnp.int32)
    gi, gj = n // N, n % N
    boff = (jnp.arange(B, dtype=jnp.int32) * N2)[:, None]
    idx_list, deg = [], jnp.zeros((N2,), f32)
    for di, dj in _HEX_OFFSETS:
        ii, jj = gi + di, gj + dj
        valid = (ii >= 0) & (ii < N) & (jj >= 0) & (jj < N)
        nb = ii * N + jj
        deg = deg + valid.astype(f32)
        per_b = jnp.where(valid[None, :], nb[None, :] + boff, zero_row)
        idx_list.append(per_b.reshape(-1))
    nbr = jnp.stack(idx_list).astype(jnp.int32)
    nbr = jnp.pad(nbr, ((0, 0), (0, pad)), constant_values=zero_row)
    nbr = nbr.reshape(6, num_chunks, _G)
    inv_deg = (1.0 / jnp.clip(deg, 1.0, None))[:, None]

    row2 = lambda a: a.reshape(1, -1)

    h = pl.pallas_call(
        functools.partial(_embed_body, B),
        grid=(B + 1,),
        in_specs=[
            pl.BlockSpec((1, N2, C), lambda b: (jnp.minimum(b, B - 1), 0, 0)),
            pl.BlockSpec((C, H), lambda b: (0, 0)),
            pl.BlockSpec((1, H), lambda b: (0, 0)),
            pl.BlockSpec((1, H), lambda b: (0, 0)),
            pl.BlockSpec((1, H), lambda b: (0, 0)),
        ],
        out_specs=pl.BlockSpec((N2, H), lambda b: (b, 0)),
        out_shape=jax.ShapeDtypeStruct((tab_rows, H), f32),
    )(xt, params['Wi'], row2(params['bi']), row2(params['ing']),
      row2(params['inb']))

    layer_call = pl.pallas_call(
        _layer_body,
        grid=(B,),
        in_specs=[
            pl.BlockSpec((N2, H), lambda b: (b, 0)),
            pl.BlockSpec((N2, H), lambda b: (b, 0)),
            pl.BlockSpec((H, H), lambda b: (0, 0)),
            pl.BlockSpec((H, H), lambda b: (0, 0)),
            pl.BlockSpec((H, H), lambda b: (0, 0)),
            pl.BlockSpec((1, H), lambda b: (0, 0)),
            pl.BlockSpec((1, H), lambda b: (0, 0)),
            pl.BlockSpec((1, H), lambda b: (0, 0)),
            pl.BlockSpec((N2, 1), lambda b: (0, 0)),
        ],
        out_specs=pl.BlockSpec((N2, H), lambda b: (b, 0)),
        out_shape=jax.ShapeDtypeStruct((tab_rows, H), f32),
        input_output_aliases={0: 0},
    )

    for lp in params['layers']:
        hs = _sc_neighbor_sum(h, nbr, num_chunks, H)
        hs = hs.reshape(num_chunks * _G, H)
        h = layer_call(h, hs, lp['Wm'], lp['Wu'][:H], lp['Wu'][H:],
                       row2(lp['bu']), row2(lp['ng']), row2(lp['nb']),
                       inv_deg)

    p3, v, t = pl.pallas_call(
        _heads_body,
        grid=(B,),
        in_specs=[
            pl.BlockSpec((N2, H), lambda b: (b, 0)),
            pl.BlockSpec((H, 1), lambda b: (0, 0)),
            pl.BlockSpec((1, 1), lambda b: (0, 0)),
            pl.BlockSpec((H, 1), lambda b: (0, 0)),
            pl.BlockSpec((1, 1), lambda b: (0, 0)),
            pl.BlockSpec((H, 256), lambda b: (0, 0)),
            pl.BlockSpec((1, 256), lambda b: (0, 0)),
            pl.BlockSpec((256, 1), lambda b: (0, 0)),
            pl.BlockSpec((1, 1), lambda b: (0, 0)),
            pl.BlockSpec((H, 4), lambda b: (0, 0)),
            pl.BlockSpec((1, 4), lambda b: (0, 0)),
        ],
        out_specs=[
            pl.BlockSpec((1, N2, 1), lambda b: (b, 0, 0)),
            pl.BlockSpec((1, 1, 1), lambda b: (b, 0, 0)),
            pl.BlockSpec((1, 1, 4), lambda b: (b, 0, 0)),
        ],
        out_shape=[
            jax.ShapeDtypeStruct((B, N2, 1), f32),
            jax.ShapeDtypeStruct((B, 1, 1), f32),
            jax.ShapeDtypeStruct((B, 1, 4), f32),
        ],
    )(h, params['Wq'], row2(params['bq']), params['Wp'], row2(params['bp']),
      params['W1'], row2(params['b1']), params['W2'], row2(params['b2']),
      params['Wt'], row2(params['bt']))

    return (p3[..., 0], v[:, 0, :], t[:, 0, :])


# speedup vs baseline: 16.2644x; 1.0367x over previous
import functools

import jax
import jax.numpy as jnp
from jax import lax
from jax.experimental import pallas as pl
from jax.experimental.pallas import tpu as pltpu
from jax.experimental.pallas import tpu_sc as plsc

_HEX_OFFSETS = ((1, 0), (-1, 0), (0, 1), (0, -1), (1, -1), (-1, 1))
_NC, _NS = 2, 16
_NW = _NC * _NS
_G = 128

_SQRT_HALF = 0.7071067811865476


def _gelu(v):
    return v * 0.5 * (1.0 + lax.erf(v * _SQRT_HALF))


def _ln(v, g, b):
    m = jnp.mean(v, axis=-1, keepdims=True)
    c = v - m
    var = jnp.mean(c * c, axis=-1, keepdims=True)
    return c * lax.rsqrt(var + 1e-5) * g + b


def _mm(a, w):
    return lax.dot_general(a, w, (((1,), (0,)), ((), ())),
                           preferred_element_type=jnp.float32)


def _embed_body(B, xt_ref, wi_ref, bi_ref, g_ref, b_ref, out_ref):
    bi = pl.program_id(0)

    @pl.when(bi == B)
    def _pad():
        out_ref[...] = jnp.zeros_like(out_ref)

    @pl.when(bi < B)
    def _emb():
        h0 = _mm(xt_ref[0], wi_ref[...]) + bi_ref[...]
        out_ref[...] = _ln(_gelu(h0), g_ref[...], b_ref[...])


def _layer_body(h_ref, hs_ref, wm_ref, wu1_ref, wu2_ref, bu_ref, g_ref,
                b_ref, invdeg_ref, out_ref):
    h = h_ref[...]
    agg = _mm(hs_ref[...], wm_ref[...]) * invdeg_ref[...]
    o = _mm(h, wu1_ref[...]) + _mm(agg, wu2_ref[...]) + bu_ref[...]
    out_ref[...] = _ln(_gelu(o) + h, g_ref[...], b_ref[...])


def _heads_body(h_ref, wq_ref, bq_ref, wp_ref, bp_ref, w1_ref, b1_ref,
                w2_ref, b2_ref, wt_ref, bt_ref, p_ref, v_ref, t_ref):
    h = h_ref[...]
    p_ref[...] = (_mm(h, wp_ref[...]) + bp_ref[...])[None]
    s = _mm(h, wq_ref[...]) + bq_ref[...]
    e = jnp.exp(s - jnp.max(s))
    a = e / jnp.sum(e)
    gf = jnp.sum(h * a, axis=0, keepdims=True)
    v1 = _gelu(_mm(gf, w1_ref[...]) + b1_ref[...])
    v_ref[...] = jnp.tanh(_mm(v1, w2_ref[...]) + b2_ref[...])[None]
    t_ref[...] = (_mm(gf, wt_ref[...]) + bt_ref[...])[None]


@functools.partial(jax.jit, static_argnums=(2, 3))
def _sc_neighbor_sum(h_tab, nbr, num_chunks, H):
    mesh = plsc.VectorSubcoreMesh(core_axis_name="c", subcore_axis_name="s")
    nseg = H // 16
    outer = (num_chunks + _NW - 1) // _NW

    @functools.partial(
        pl.kernel, mesh=mesh,
        out_type=jax.ShapeDtypeStruct((num_chunks, _G, H), jnp.float32),
        scratch_types=[pltpu.VMEM((6, _G), jnp.int32)]
        + [pltpu.VMEM((_G, H), jnp.float32) for _ in range(6)]
        + [pltpu.SemaphoreType.DMA],
    )
    def k(h_hbm, nbr_hbm, out_hbm, idxs,
          r0, r1, r2, r3, r4, r5, sem):
        rows = (r0, r1, r2, r3, r4, r5)
        wid = lax.axis_index("s") * _NC + lax.axis_index("c")

        def chunk_body(ci, carry):
            c = wid + ci * _NW

            @pl.when(c < num_chunks)
            def _work():
                pltpu.sync_copy(nbr_hbm.at[c], idxs)
                cps = [pltpu.async_copy(h_hbm.at[idxs.at[o]], rows[o], sem)
                       for o in range(6)]
                for cp in cps:
                    cp.wait()

                def row_body(rr, rcarry):
                    for ss in range(nseg):
                        sl = pl.ds(ss * 16, 16)
                        r0[rr, sl] = ((r0[rr, sl] + r1[rr, sl])
                                      + (r2[rr, sl] + r3[rr, sl])
                                      + (r4[rr, sl] + r5[rr, sl]))
                    return rcarry

                lax.fori_loop(0, _G, row_body, 0)
                pltpu.sync_copy(r0, out_hbm.at[c])
            return carry

        lax.fori_loop(0, outer, chunk_body, 0)

    return k(h_tab, nbr)


def kernel(x, params, edge_index):
    B, C, N, _ = x.shape
    N2 = N * N
    H = params['Wi'].shape[1]
    total = B * N2
    zero_row = total
    tab_rows = (B + 1) * N2
    num_chunks = (total + _G - 1) // _G
    pad = num_chunks * _G - total

    f32 = jnp.float32
    xt = jnp.transpose(x.reshape(B, C, N2), (0, 2, 1))

    n = jnp.arange(N2, dtype=jnp.int32)
    gi, gj = n // N, n % N
    boff = (jnp.arange(B, dtype=jnp.int32) * N2)[:, None]
    idx_list, deg = [], jnp.zeros((N2,), f32)
    for di, dj in _HEX_OFFSETS:
        ii, jj = gi + di, gj + dj
        valid = (ii >= 0) & (ii < N) & (jj >= 0) & (jj < N)
        nb = ii * N + jj
        deg = deg + valid.astype(f32)
        per_b = jnp.where(valid[None, :], nb[None, :] + boff, zero_row)
        idx_list.append(per_b.reshape(-1))
    nbr = jnp.stack(idx_list).astype(jnp.int32)
    nbr = jnp.pad(nbr, ((0, 0), (0, pad)), constant_values=zero_row)
    nbr = jnp.transpose(nbr.reshape(6, num_chunks, _G), (1, 0, 2))
    inv_deg = (1.0 / jnp.clip(deg, 1.0, None))[:, None]

    row2 = lambda a: a.reshape(1, -1)

    h = pl.pallas_call(
        functools.partial(_embed_body, B),
        grid=(B + 1,),
        in_specs=[
            pl.BlockSpec((1, N2, C), lambda b: (jnp.minimum(b, B - 1), 0, 0)),
            pl.BlockSpec((C, H), lambda b: (0, 0)),
            pl.BlockSpec((1, H), lambda b: (0, 0)),
            pl.BlockSpec((1, H), lambda b: (0, 0)),
            pl.BlockSpec((1, H), lambda b: (0, 0)),
        ],
        out_specs=pl.BlockSpec((N2, H), lambda b: (b, 0)),
        out_shape=jax.ShapeDtypeStruct((tab_rows, H), f32),
    )(xt, params['Wi'], row2(params['bi']), row2(params['ing']),
      row2(params['inb']))

    layer_call = pl.pallas_call(
        _layer_body,
        grid=(B,),
        in_specs=[
            pl.BlockSpec((N2, H), lambda b: (b, 0)),
            pl.BlockSpec((N2, H), lambda b: (b, 0)),
            pl.BlockSpec((H, H), lambda b: (0, 0)),
            pl.BlockSpec((H, H), lambda b: (0, 0)),
            pl.BlockSpec((H, H), lambda b: (0, 0)),
            pl.BlockSpec((1, H), lambda b: (0, 0)),
            pl.BlockSpec((1, H), lambda b: (0, 0)),
            pl.BlockSpec((1, H), lambda b: (0, 0)),
            pl.BlockSpec((N2, 1), lambda b: (0, 0)),
        ],
        out_specs=pl.BlockSpec((N2, H), lambda b: (b, 0)),
        out_shape=jax.ShapeDtypeStruct((tab_rows, H), f32),
        input_output_aliases={0: 0},
    )

    for lp in params['layers']:
        hs = _sc_neighbor_sum(h, nbr, num_chunks, H)
        hs = hs.reshape(num_chunks * _G, H)
        h = layer_call(h, hs, lp['Wm'], lp['Wu'][:H], lp['Wu'][H:],
                       row2(lp['bu']), row2(lp['ng']), row2(lp['nb']),
                       inv_deg)

    p3, v, t = pl.pallas_call(
        _heads_body,
        grid=(B,),
        in_specs=[
            pl.BlockSpec((N2, H), lambda b: (b, 0)),
            pl.BlockSpec((H, 1), lambda b: (0, 0)),
            pl.BlockSpec((1, 1), lambda b: (0, 0)),
            pl.BlockSpec((H, 1), lambda b: (0, 0)),
            pl.BlockSpec((1, 1), lambda b: (0, 0)),
            pl.BlockSpec((H, 256), lambda b: (0, 0)),
            pl.BlockSpec((1, 256), lambda b: (0, 0)),
            pl.BlockSpec((256, 1), lambda b: (0, 0)),
            pl.BlockSpec((1, 1), lambda b: (0, 0)),
            pl.BlockSpec((H, 4), lambda b: (0, 0)),
            pl.BlockSpec((1, 4), lambda b: (0, 0)),
        ],
        out_specs=[
            pl.BlockSpec((1, N2, 1), lambda b: (b, 0, 0)),
            pl.BlockSpec((1, 1, 1), lambda b: (b, 0, 0)),
            pl.BlockSpec((1, 1, 4), lambda b: (b, 0, 0)),
        ],
        out_shape=[
            jax.ShapeDtypeStruct((B, N2, 1), f32),
            jax.ShapeDtypeStruct((B, 1, 1), f32),
            jax.ShapeDtypeStruct((B, 1, 4), f32),
        ],
    )(h, params['Wq'], row2(params['bq']), params['Wp'], row2(params['bp']),
      params['W1'], row2(params['b1']), params['W2'], row2(params['b2']),
      params['Wt'], row2(params['bt']))

    return (p3[..., 0], v[:, 0, :], t[:, 0, :])


# speedup vs baseline: 50.5307x; 3.1068x over previous
import functools

import jax
import jax.numpy as jnp
from jax import lax
from jax.experimental import pallas as pl
from jax.experimental.pallas import tpu as pltpu
from jax.experimental.pallas import tpu_sc as plsc

_NC, _NS = 2, 16
_NW = _NC * _NS
_G = 128

_SQRT_HALF = 0.7071067811865476


def _gelu(v):
    return v * 0.5 * (1.0 + lax.erf(v * _SQRT_HALF))


def _ln(v, g, b):
    m = jnp.mean(v, axis=-1, keepdims=True)
    c = v - m
    var = jnp.mean(c * c, axis=-1, keepdims=True)
    return c * lax.rsqrt(var + 1e-5) * g + b


def _mm(a, w):
    return lax.dot_general(a, w, (((1,), (0,)), ((), ())),
                           preferred_element_type=jnp.float32)


def _embed_body(B, xt_ref, wi_ref, bi_ref, g_ref, b_ref, out_ref):
    bi = pl.program_id(0)

    @pl.when((bi == 0) | (bi == B + 1))
    def _pad():
        out_ref[...] = jnp.zeros_like(out_ref)

    @pl.when((bi > 0) & (bi <= B))
    def _emb():
        h0 = _mm(xt_ref[0], wi_ref[...]) + bi_ref[...]
        out_ref[...] = _ln(_gelu(h0), g_ref[...], b_ref[...])


def _layer_body(h_ref, hs_ref, wm_ref, wu1_ref, wu2_ref, bu_ref, g_ref,
                b_ref, invdeg_ref, out_ref):
    h = h_ref[...]
    agg = _mm(hs_ref[...], wm_ref[...]) * invdeg_ref[...]
    o = _mm(h, wu1_ref[...]) + _mm(agg, wu2_ref[...]) + bu_ref[...]
    out_ref[...] = _ln(_gelu(o) + h, g_ref[...], b_ref[...])


def _heads_body(h_ref, wq_ref, bq_ref, wp_ref, bp_ref, w1_ref, b1_ref,
                w2_ref, b2_ref, wt_ref, bt_ref, p_ref, v_ref, t_ref):
    h = h_ref[...]
    p_ref[...] = (_mm(h, wp_ref[...]) + bp_ref[...])[None]
    s = _mm(h, wq_ref[...]) + bq_ref[...]
    e = jnp.exp(s - jnp.max(s))
    a = e / jnp.sum(e)
    gf = jnp.sum(h * a, axis=0, keepdims=True)
    v1 = _gelu(_mm(gf, w1_ref[...]) + b1_ref[...])
    v_ref[...] = jnp.tanh(_mm(v1, w2_ref[...]) + b2_ref[...])[None]
    t_ref[...] = (_mm(gf, wt_ref[...]) + bt_ref[...])[None]


def _sc_neighbor_sum(h_tab, num_chunks, N, H):
    mesh = plsc.VectorSubcoreMesh(core_axis_name="c", subcore_axis_name="s")
    N2 = N * N
    nseg = H // 16
    al = (N2 - N) % 8
    win = -(-(_G + 2 * N + al) // 8) * 8
    outer = (num_chunks + _NW - 1) // _NW
    assert outer % 2 == 0

    @functools.partial(
        pl.kernel, mesh=mesh,
        out_type=jax.ShapeDtypeStruct((num_chunks, _G, H), jnp.float32),
        scratch_types=[pltpu.VMEM((win, H), jnp.float32),
                       pltpu.VMEM((win, H), jnp.float32),
                       pltpu.VMEM((_G, H), jnp.float32),
                       pltpu.SemaphoreType.DMA,
                       pltpu.SemaphoreType.DMA],
    )
    def k(h_hbm, out_hbm, w0, w1, acc, s0, s1):
        ws = (w0, w1)
        sems = (s0, s1)
        wid = lax.axis_index("s") * _NC + lax.axis_index("c")

        def fire(c, buf, sem):
            start = N2 + c * _G - N - al
            return pltpu.async_copy(h_hbm.at[pl.ds(start, win)], buf, sem)

        @pl.when(wid < num_chunks)
        def _pro():
            fire(wid, ws[0], sems[0])

        def outer_body(ci2, carry):
            for b in range(2):
                ci = ci2 * 2 + b
                c = wid + ci * _NW
                cn = c + _NW

                @pl.when(cn < num_chunks)
                def _fire_next():
                    fire(cn, ws[1 - b], sems[1 - b])

                @pl.when(c < num_chunks)
                def _work():
                    w = ws[b]
                    pltpu.make_async_copy(
                        h_hbm.at[pl.ds(0, win)], w, sems[b]).wait()
                    base = c * _G

                    def row_body(rr, rcarry):
                        g = base + rr
                        n = lax.rem(g, N2)
                        i = n // N
                        j = lax.rem(n, N)
                        dn = (i > 0).astype(jnp.float32)
                        up = (i < N - 1).astype(jnp.float32)
                        lf = (j > 0).astype(jnp.float32)
                        rt = (j < N - 1).astype(jnp.float32)
                        m_p1 = rt
                        m_m1 = lf
                        m_pN = up
                        m_mN = dn
                        m_pD = up * lf
                        m_mD = dn * rt
                        ra = rr + al
                        for ss in range(nseg):
                            sl = pl.ds(ss * 16, 16)
                            acc[rr, sl] = (
                                m_pN * w[ra + 2 * N, sl]
                                + m_mN * w[ra, sl]
                                + m_p1 * w[ra + N + 1, sl]
                                + m_m1 * w[ra + N - 1, sl]
                                + m_pD * w[ra + 2 * N - 1, sl]
                                + m_mD * w[ra + 1, sl])
                        return rcarry

                    lax.fori_loop(0, _G, row_body, 0)
                    pltpu.sync_copy(acc, out_hbm.at[c])
            return carry

        lax.fori_loop(0, outer // 2, outer_body, 0)

    return k(h_tab)


def kernel(x, params, edge_index):
    B, C, N, _ = x.shape
    N2 = N * N
    H = params['Wi'].shape[1]
    total = B * N2
    num_chunks = (total + _G - 1) // _G

    f32 = jnp.float32
    xt = jnp.transpose(x.reshape(B, C, N2), (0, 2, 1))

    n = jnp.arange(N2, dtype=jnp.int32)
    gi, gj = n // N, n % N
    deg = jnp.zeros((N2,), f32)
    for di, dj in ((1, 0), (-1, 0), (0, 1), (0, -1), (1, -1), (-1, 1)):
        ii, jj = gi + di, gj + dj
        valid = (ii >= 0) & (ii < N) & (jj >= 0) & (jj < N)
        deg = deg + valid.astype(f32)
    inv_deg = (1.0 / jnp.clip(deg, 1.0, None))[:, None]

    row2 = lambda a: a.reshape(1, -1)
    tab_rows = (B + 2) * N2

    h = pl.pallas_call(
        functools.partial(_embed_body, B),
        grid=(B + 2,),
        in_specs=[
            pl.BlockSpec((1, N2, C),
                         lambda b: (jnp.clip(b - 1, 0, B - 1), 0, 0)),
            pl.BlockSpec((C, H), lambda b: (0, 0)),
            pl.BlockSpec((1, H), lambda b: (0, 0)),
            pl.BlockSpec((1, H), lambda b: (0, 0)),
            pl.BlockSpec((1, H), lambda b: (0, 0)),
        ],
        out_specs=pl.BlockSpec((N2, H), lambda b: (b, 0)),
        out_shape=jax.ShapeDtypeStruct((tab_rows, H), f32),
    )(xt, params['Wi'], row2(params['bi']), row2(params['ing']),
      row2(params['inb']))

    layer_call = pl.pallas_call(
        _layer_body,
        grid=(B,),
        in_specs=[
            pl.BlockSpec((N2, H), lambda b: (b + 1, 0)),
            pl.BlockSpec((N2, H), lambda b: (b, 0)),
            pl.BlockSpec((H, H), lambda b: (0, 0)),
            pl.BlockSpec((H, H), lambda b: (0, 0)),
            pl.BlockSpec((H, H), lambda b: (0, 0)),
            pl.BlockSpec((1, H), lambda b: (0, 0)),
            pl.BlockSpec((1, H), lambda b: (0, 0)),
            pl.BlockSpec((1, H), lambda b: (0, 0)),
            pl.BlockSpec((N2, 1), lambda b: (0, 0)),
        ],
        out_specs=pl.BlockSpec((N2, H), lambda b: (b + 1, 0)),
        out_shape=jax.ShapeDtypeStruct((tab_rows, H), f32),
        input_output_aliases={0: 0},
    )

    for lp in params['layers']:
        hs = _sc_neighbor_sum(h, num_chunks, N, H)
        hs = hs.reshape(num_chunks * _G, H)
        h = layer_call(h, hs, lp['Wm'], lp['Wu'][:H], lp['Wu'][H:],
                       row2(lp['bu']), row2(lp['ng']), row2(lp['nb']),
                       inv_deg)

    p3, v, t = pl.pallas_call(
        _heads_body,
        grid=(B,),
        in_specs=[
            pl.BlockSpec((N2, H), lambda b: (b + 1, 0)),
            pl.BlockSpec((H, 1), lambda b: (0, 0)),
            pl.BlockSpec((1, 1), lambda b: (0, 0)),
            pl.BlockSpec((H, 1), lambda b: (0, 0)),
            pl.BlockSpec((1, 1), lambda b: (0, 0)),
            pl.BlockSpec((H, 256), lambda b: (0, 0)),
            pl.BlockSpec((1, 256), lambda b: (0, 0)),
            pl.BlockSpec((256, 1), lambda b: (0, 0)),
            pl.BlockSpec((1, 1), lambda b: (0, 0)),
            pl.BlockSpec((H, 4), lambda b: (0, 0)),
            pl.BlockSpec((1, 4), lambda b: (0, 0)),
        ],
        out_specs=[
            pl.BlockSpec((1, N2, 1), lambda b: (b, 0, 0)),
            pl.BlockSpec((1, 1, 1), lambda b: (b, 0, 0)),
            pl.BlockSpec((1, 1, 4), lambda b: (b, 0, 0)),
        ],
        out_shape=[
            jax.ShapeDtypeStruct((B, N2, 1), f32),
            jax.ShapeDtypeStruct((B, 1, 1), f32),
            jax.ShapeDtypeStruct((B, 1, 4), f32),
        ],
    )(h, params['Wq'], row2(params['bq']), params['Wp'], row2(params['bp']),
      params['W1'], row2(params['b1']), params['W2'], row2(params['b2']),
      params['Wt'], row2(params['bt']))

    return (p3[..., 0], v[:, 0, :], t[:, 0, :])


# speedup vs baseline: 58.7576x; 1.1628x over previous
import functools

import jax
import jax.numpy as jnp
from jax import lax
from jax.experimental import pallas as pl
from jax.experimental.pallas import tpu as pltpu
from jax.experimental.pallas import tpu_sc as plsc

_NC, _NS = 2, 16
_NW = _NC * _NS
_G = 160

_SQRT_HALF = 0.7071067811865476


def _gelu(v):
    return v * 0.5 * (1.0 + lax.erf(v * _SQRT_HALF))


def _ln(v, g, b):
    m = jnp.mean(v, axis=-1, keepdims=True)
    c = v - m
    var = jnp.mean(c * c, axis=-1, keepdims=True)
    return c * lax.rsqrt(var + 1e-5) * g + b


def _mm(a, w):
    return lax.dot_general(a, w, (((1,), (0,)), ((), ())),
                           preferred_element_type=jnp.float32)


def _embed_body(B, xt_ref, wi_ref, bi_ref, g_ref, b_ref, out_ref):
    bi = pl.program_id(0)

    @pl.when((bi == 0) | (bi == B + 1))
    def _pad():
        out_ref[...] = jnp.zeros_like(out_ref)

    @pl.when((bi > 0) & (bi <= B))
    def _emb():
        h0 = _mm(xt_ref[0], wi_ref[...]) + bi_ref[...]
        out_ref[...] = _ln(_gelu(h0), g_ref[...], b_ref[...])


def _layer_body(h_ref, hs_ref, wm_ref, wu1_ref, wu2_ref, bu_ref, g_ref,
                b_ref, invdeg_ref, out_ref):
    h = h_ref[...]
    agg = _mm(hs_ref[...], wm_ref[...]) * invdeg_ref[...]
    o = _mm(h, wu1_ref[...]) + _mm(agg, wu2_ref[...]) + bu_ref[...]
    out_ref[...] = _ln(_gelu(o) + h, g_ref[...], b_ref[...])


def _heads_body(h_ref, wq_ref, bq_ref, wp_ref, bp_ref, w1_ref, b1_ref,
                w2_ref, b2_ref, wt_ref, bt_ref, p_ref, v_ref, t_ref):
    h = h_ref[...]
    p_ref[...] = (_mm(h, wp_ref[...]) + bp_ref[...])[None]
    s = _mm(h, wq_ref[...]) + bq_ref[...]
    e = jnp.exp(s - jnp.max(s))
    a = e / jnp.sum(e)
    gf = jnp.sum(h * a, axis=0, keepdims=True)
    v1 = _gelu(_mm(gf, w1_ref[...]) + b1_ref[...])
    v_ref[...] = jnp.tanh(_mm(v1, w2_ref[...]) + b2_ref[...])[None]
    t_ref[...] = (_mm(gf, wt_ref[...]) + bt_ref[...])[None]


def _sc_neighbor_sum(h_tab, num_chunks, N, H):
    mesh = plsc.VectorSubcoreMesh(core_axis_name="c", subcore_axis_name="s")
    N2 = N * N
    nseg = H // 16
    al = (N2 - N) % 8
    win = -(-(_G + 2 * N + al) // 8) * 8
    outer = (num_chunks + _NW - 1) // _NW
    assert outer % 2 == 0

    @functools.partial(
        pl.kernel, mesh=mesh,
        out_type=jax.ShapeDtypeStruct((num_chunks, _G, H), jnp.float32),
        scratch_types=[pltpu.VMEM((win, H), jnp.float32),
                       pltpu.VMEM((win, H), jnp.float32),
                       pltpu.VMEM((_G, H), jnp.float32),
                       pltpu.SemaphoreType.DMA,
                       pltpu.SemaphoreType.DMA],
    )
    def k(h_hbm, out_hbm, w0, w1, acc, s0, s1):
        ws = (w0, w1)
        sems = (s0, s1)
        wid = lax.axis_index("s") * _NC + lax.axis_index("c")

        def fire(c, buf, sem):
            start = N2 + c * _G - N - al
            return pltpu.async_copy(h_hbm.at[pl.ds(start, win)], buf, sem)

        @pl.when(wid < num_chunks)
        def _pro():
            fire(wid, ws[0], sems[0])

        def outer_body(ci2, carry):
            for b in range(2):
                ci = ci2 * 2 + b
                c = wid + ci * _NW
                cn = c + _NW

                @pl.when(cn < num_chunks)
                def _fire_next():
                    fire(cn, ws[1 - b], sems[1 - b])

                @pl.when(c < num_chunks)
                def _work():
                    w = ws[b]
                    pltpu.make_async_copy(
                        h_hbm.at[pl.ds(0, win)], w, sems[b]).wait()
                    base = c * _G

                    def row_body(rr, rcarry):
                        g = base + rr
                        n = lax.rem(g, N2)
                        i = n // N
                        j = lax.rem(n, N)
                        dn = (i > 0).astype(jnp.float32)
                        up = (i < N - 1).astype(jnp.float32)
                        lf = (j > 0).astype(jnp.float32)
                        rt = (j < N - 1).astype(jnp.float32)
                        m_p1 = rt
                        m_m1 = lf
                        m_pN = up
                        m_mN = dn
                        m_pD = up * lf
                        m_mD = dn * rt
                        ra = rr + al
                        for ss in range(nseg):
                            sl = pl.ds(ss * 16, 16)
                            acc[rr, sl] = (
                                m_pN * w[ra + 2 * N, sl]
                                + m_mN * w[ra, sl]
                                + m_p1 * w[ra + N + 1, sl]
                                + m_m1 * w[ra + N - 1, sl]
                                + m_pD * w[ra + 2 * N - 1, sl]
                                + m_mD * w[ra + 1, sl])
                        return rcarry

                    lax.fori_loop(0, _G, row_body, 0)
                    pltpu.sync_copy(acc, out_hbm.at[c])
            return carry

        lax.fori_loop(0, outer // 2, outer_body, 0)

    return k(h_tab)


def kernel(x, params, edge_index):
    B, C, N, _ = x.shape
    N2 = N * N
    H = params['Wi'].shape[1]
    GB = B // 2
    total = GB * N2
    num_chunks = (total + _G - 1) // _G

    f32 = jnp.float32
    xt = jnp.transpose(x.reshape(B, C, N2), (0, 2, 1))

    n = jnp.arange(N2, dtype=jnp.int32)
    gi, gj = n // N, n % N
    deg = jnp.zeros((N2,), f32)
    for di, dj in ((1, 0), (-1, 0), (0, 1), (0, -1), (1, -1), (-1, 1)):
        ii, jj = gi + di, gj + dj
        valid = (ii >= 0) & (ii < N) & (jj >= 0) & (jj < N)
        deg = deg + valid.astype(f32)
    inv_deg = (1.0 / jnp.clip(deg, 1.0, None))[:, None]

    row2 = lambda a: a.reshape(1, -1)
    tab_rows = (GB + 2) * N2

    def embed_group(g):
        return pl.pallas_call(
            functools.partial(_embed_body, GB),
            grid=(GB + 2,),
            in_specs=[
                pl.BlockSpec(
                    (1, N2, C),
                    lambda b: (jnp.clip(b - 1, 0, GB - 1) + g * GB, 0, 0)),
                pl.BlockSpec((C, H), lambda b: (0, 0)),
                pl.BlockSpec((1, H), lambda b: (0, 0)),
                pl.BlockSpec((1, H), lambda b: (0, 0)),
                pl.BlockSpec((1, H), lambda b: (0, 0)),
            ],
            out_specs=pl.BlockSpec((N2, H), lambda b: (b, 0)),
            out_shape=jax.ShapeDtypeStruct((tab_rows, H), f32),
        )(xt, params['Wi'], row2(params['bi']), row2(params['ing']),
          row2(params['inb']))

    hA, hB = embed_group(0), embed_group(1)

    layer_call = pl.pallas_call(
        _layer_body,
        grid=(GB,),
        in_specs=[
            pl.BlockSpec((N2, H), lambda b: (b + 1, 0)),
            pl.BlockSpec((N2, H), lambda b: (b, 0)),
            pl.BlockSpec((H, H), lambda b: (0, 0)),
            pl.BlockSpec((H, H), lambda b: (0, 0)),
            pl.BlockSpec((H, H), lambda b: (0, 0)),
            pl.BlockSpec((1, H), lambda b: (0, 0)),
            pl.BlockSpec((1, H), lambda b: (0, 0)),
            pl.BlockSpec((1, H), lambda b: (0, 0)),
            pl.BlockSpec((N2, 1), lambda b: (0, 0)),
        ],
        out_specs=pl.BlockSpec((N2, H), lambda b: (b + 1, 0)),
        out_shape=jax.ShapeDtypeStruct((tab_rows, H), f32),
        input_output_aliases={0: 0},
    )

    for lp in params['layers']:
        hsA = _sc_neighbor_sum(hA, num_chunks, N, H).reshape(-1, H)
        hsB = _sc_neighbor_sum(hB, num_chunks, N, H).reshape(-1, H)
        wu1, wu2 = lp['Wu'][:H], lp['Wu'][H:]
        hA = layer_call(hA, hsA, lp['Wm'], wu1, wu2, row2(lp['bu']),
                        row2(lp['ng']), row2(lp['nb']), inv_deg)
        hB = layer_call(hB, hsB, lp['Wm'], wu1, wu2, row2(lp['bu']),
                        row2(lp['ng']), row2(lp['nb']), inv_deg)

    heads_call = pl.pallas_call(
        _heads_body,
        grid=(GB,),
        in_specs=[
            pl.BlockSpec((N2, H), lambda b: (b + 1, 0)),
            pl.BlockSpec((H, 1), lambda b: (0, 0)),
            pl.BlockSpec((1, 1), lambda b: (0, 0)),
            pl.BlockSpec((H, 1), lambda b: (0, 0)),
            pl.BlockSpec((1, 1), lambda b: (0, 0)),
            pl.BlockSpec((H, 256), lambda b: (0, 0)),
            pl.BlockSpec((1, 256), lambda b: (0, 0)),
            pl.BlockSpec((256, 1), lambda b: (0, 0)),
            pl.BlockSpec((1, 1), lambda b: (0, 0)),
            pl.BlockSpec((H, 4), lambda b: (0, 0)),
            pl.BlockSpec((1, 4), lambda b: (0, 0)),
        ],
        out_specs=[
            pl.BlockSpec((1, N2, 1), lambda b: (b, 0, 0)),
            pl.BlockSpec((1, 1, 1), lambda b: (b, 0, 0)),
            pl.BlockSpec((1, 1, 4), lambda b: (b, 0, 0)),
        ],
        out_shape=[
            jax.ShapeDtypeStruct((GB, N2, 1), f32),
            jax.ShapeDtypeStruct((GB, 1, 1), f32),
            jax.ShapeDtypeStruct((GB, 1, 4), f32),
        ],
    )
    hw = (params['Wq'], row2(params['bq']), params['Wp'], row2(params['bp']),
          params['W1'], row2(params['b1']), params['W2'], row2(params['b2']),
          params['Wt'], row2(params['bt']))
    pA, vA, tA = heads_call(hA, *hw)
    pB, vB, tB = heads_call(hB, *hw)

    p = jnp.concatenate([pA[..., 0], pB[..., 0]], axis=0)
    v = jnp.concatenate([vA[:, 0, :], vB[:, 0, :]], axis=0)
    t = jnp.concatenate([tA[:, 0, :], tB[:, 0, :]], axis=0)
    return (p, v, t)


# speedup vs baseline: 62.7079x; 1.0672x over previous
import functools

import jax
import jax.numpy as jnp
from jax import lax
from jax.experimental import pallas as pl
from jax.experimental.pallas import tpu as pltpu
from jax.experimental.pallas import tpu_sc as plsc

_NC, _NS = 2, 16
_NW = _NC * _NS
_G = 160

_SQRT_HALF = 0.7071067811865476


def _gelu(v):
    return v * 0.5 * (1.0 + lax.erf(v * _SQRT_HALF))


def _ln(v, g, b):
    m = jnp.mean(v, axis=-1, keepdims=True)
    c = v - m
    var = jnp.mean(c * c, axis=-1, keepdims=True)
    return c * lax.rsqrt(var + 1e-5) * g + b


def _mm(a, w):
    return lax.dot_general(a, w, (((1,), (0,)), ((), ())),
                           preferred_element_type=jnp.float32)


def _embed_body(B, xt_ref, wi_ref, bi_ref, g_ref, b_ref, out_ref):
    bi = pl.program_id(0)

    @pl.when((bi == 0) | (bi == B + 1))
    def _pad():
        out_ref[...] = jnp.zeros_like(out_ref)

    @pl.when((bi > 0) & (bi <= B))
    def _emb():
        h0 = _mm(xt_ref[0], wi_ref[...]) + bi_ref[...]
        out_ref[...] = _ln(_gelu(h0), g_ref[...], b_ref[...])


def _layer_body(h_ref, hs_ref, wm_ref, wu1_ref, wu2_ref, bu_ref, g_ref,
                b_ref, invdeg_ref, out_ref):
    h = h_ref[...]
    agg = _mm(hs_ref[...], wm_ref[...]) * invdeg_ref[...]
    o = _mm(h, wu1_ref[...]) + _mm(agg, wu2_ref[...]) + bu_ref[...]
    out_ref[...] = _ln(_gelu(o) + h, g_ref[...], b_ref[...])


def _heads_body(h_ref, wq_ref, bq_ref, wp_ref, bp_ref, w1_ref, b1_ref,
                w2_ref, b2_ref, wt_ref, bt_ref, p_ref, v_ref, t_ref):
    h = h_ref[...]
    p_ref[...] = (_mm(h, wp_ref[...]) + bp_ref[...])[None]
    s = _mm(h, wq_ref[...]) + bq_ref[...]
    e = jnp.exp(s - jnp.max(s))
    a = e / jnp.sum(e)
    gf = jnp.sum(h * a, axis=0, keepdims=True)
    v1 = _gelu(_mm(gf, w1_ref[...]) + b1_ref[...])
    v_ref[...] = jnp.tanh(_mm(v1, w2_ref[...]) + b2_ref[...])[None]
    t_ref[...] = (_mm(gf, wt_ref[...]) + bt_ref[...])[None]


def _sc_neighbor_sum(h_tab, num_chunks, N, H):
    mesh = plsc.VectorSubcoreMesh(core_axis_name="c", subcore_axis_name="s")
    N2 = N * N
    nseg = H // 16
    al = (N2 - N) % 8
    win = -(-(_G + 2 * N + al) // 8) * 8
    outer = (num_chunks + _NW - 1) // _NW
    assert outer % 2 == 0

    @functools.partial(
        pl.kernel, mesh=mesh,
        out_type=jax.ShapeDtypeStruct((num_chunks, _G, H), jnp.float32),
        scratch_types=[pltpu.VMEM((win, H), jnp.float32),
                       pltpu.VMEM((win, H), jnp.float32),
                       pltpu.VMEM((_G, H), jnp.float32),
                       pltpu.SemaphoreType.DMA,
                       pltpu.SemaphoreType.DMA],
    )
    def k(h_hbm, out_hbm, w0, w1, acc, s0, s1):
        ws = (w0, w1)
        sems = (s0, s1)
        wid = lax.axis_index("s") * _NC + lax.axis_index("c")

        def fire(c, buf, sem):
            start = N2 + c * _G - N - al
            return pltpu.async_copy(h_hbm.at[pl.ds(start, win)], buf, sem)

        @pl.when(wid < num_chunks)
        def _pro():
            fire(wid, ws[0], sems[0])

        def outer_body(ci2, carry):
            for b in range(2):
                ci = ci2 * 2 + b
                c = wid + ci * _NW
                cn = c + _NW

                @pl.when(cn < num_chunks)
                def _fire_next():
                    fire(cn, ws[1 - b], sems[1 - b])

                @pl.when(c < num_chunks)
                def _work():
                    w = ws[b]
                    pltpu.make_async_copy(
                        h_hbm.at[pl.ds(0, win)], w, sems[b]).wait()
                    base = c * _G
                    K = 8

                    def row_body(it, rcarry):
                        rb = it * K
                        ms = []
                        for rk in range(K):
                            g = base + rb + rk
                            n = lax.rem(g, N2)
                            i = n // N
                            j = lax.rem(n, N)
                            dn = (i > 0).astype(jnp.float32)
                            up = (i < N - 1).astype(jnp.float32)
                            lf = (j > 0).astype(jnp.float32)
                            rt = (j < N - 1).astype(jnp.float32)
                            ms.append((up, dn, rt, lf, up * lf, dn * rt))
                        ra0 = rb + al
                        taps = (list(range(0, K + 1))
                                + list(range(N - 1, N + K + 1))
                                + list(range(2 * N - 1, 2 * N + K)))
                        for ss in range(nseg):
                            sl = pl.ds(ss * 16, 16)
                            vals = {t: w[ra0 + t, sl] for t in taps}
                            for rk in range(K):
                                m_pN, m_mN, m_p1, m_m1, m_pD, m_mD = ms[rk]
                                acc[rb + rk, sl] = (
                                    m_pN * vals[2 * N + rk]
                                    + m_mN * vals[rk]
                                    + m_p1 * vals[N + 1 + rk]
                                    + m_m1 * vals[N - 1 + rk]
                                    + m_pD * vals[2 * N - 1 + rk]
                                    + m_mD * vals[1 + rk])
                        return rcarry

                    lax.fori_loop(0, _G // K, row_body, 0)
                    pltpu.sync_copy(acc, out_hbm.at[c])
            return carry

        lax.fori_loop(0, outer // 2, outer_body, 0)

    return k(h_tab)


def kernel(x, params, edge_index):
    B, C, N, _ = x.shape
    N2 = N * N
    H = params['Wi'].shape[1]
    GB = B // 2
    total = GB * N2
    num_chunks = (total + _G - 1) // _G

    f32 = jnp.float32
    xt = jnp.transpose(x.reshape(B, C, N2), (0, 2, 1))

    n = jnp.arange(N2, dtype=jnp.int32)
    gi, gj = n // N, n % N
    deg = jnp.zeros((N2,), f32)
    for di, dj in ((1, 0), (-1, 0), (0, 1), (0, -1), (1, -1), (-1, 1)):
        ii, jj = gi + di, gj + dj
        valid = (ii >= 0) & (ii < N) & (jj >= 0) & (jj < N)
        deg = deg + valid.astype(f32)
    inv_deg = (1.0 / jnp.clip(deg, 1.0, None))[:, None]

    row2 = lambda a: a.reshape(1, -1)
    tab_rows = (GB + 2) * N2

    def embed_group(g):
        return pl.pallas_call(
            functools.partial(_embed_body, GB),
            grid=(GB + 2,),
            in_specs=[
                pl.BlockSpec(
                    (1, N2, C),
                    lambda b: (jnp.clip(b - 1, 0, GB - 1) + g * GB, 0, 0)),
                pl.BlockSpec((C, H), lambda b: (0, 0)),
                pl.BlockSpec((1, H), lambda b: (0, 0)),
                pl.BlockSpec((1, H), lambda b: (0, 0)),
                pl.BlockSpec((1, H), lambda b: (0, 0)),
            ],
            out_specs=pl.BlockSpec((N2, H), lambda b: (b, 0)),
            out_shape=jax.ShapeDtypeStruct((tab_rows, H), f32),
        )(xt, params['Wi'], row2(params['bi']), row2(params['ing']),
          row2(params['inb']))

    hA, hB = embed_group(0), embed_group(1)

    layer_call = pl.pallas_call(
        _layer_body,
        grid=(GB,),
        in_specs=[
            pl.BlockSpec((N2, H), lambda b: (b + 1, 0)),
            pl.BlockSpec((N2, H), lambda b: (b, 0)),
            pl.BlockSpec((H, H), lambda b: (0, 0)),
            pl.BlockSpec((H, H), lambda b: (0, 0)),
            pl.BlockSpec((H, H), lambda b: (0, 0)),
            pl.BlockSpec((1, H), lambda b: (0, 0)),
            pl.BlockSpec((1, H), lambda b: (0, 0)),
            pl.BlockSpec((1, H), lambda b: (0, 0)),
            pl.BlockSpec((N2, 1), lambda b: (0, 0)),
        ],
        out_specs=pl.BlockSpec((N2, H), lambda b: (b + 1, 0)),
        out_shape=jax.ShapeDtypeStruct((tab_rows, H), f32),
        input_output_aliases={0: 0},
    )

    for lp in params['layers']:
        hsA = _sc_neighbor_sum(hA, num_chunks, N, H).reshape(-1, H)
        hsB = _sc_neighbor_sum(hB, num_chunks, N, H).reshape(-1, H)
        wu1, wu2 = lp['Wu'][:H], lp['Wu'][H:]
        hA = layer_call(hA, hsA, lp['Wm'], wu1, wu2, row2(lp['bu']),
                        row2(lp['ng']), row2(lp['nb']), inv_deg)
        hB = layer_call(hB, hsB, lp['Wm'], wu1, wu2, row2(lp['bu']),
                        row2(lp['ng']), row2(lp['nb']), inv_deg)

    heads_call = pl.pallas_call(
        _heads_body,
        grid=(GB,),
        in_specs=[
            pl.BlockSpec((N2, H), lambda b: (b + 1, 0)),
            pl.BlockSpec((H, 1), lambda b: (0, 0)),
            pl.BlockSpec((1, 1), lambda b: (0, 0)),
            pl.BlockSpec((H, 1), lambda b: (0, 0)),
            pl.BlockSpec((1, 1), lambda b: (0, 0)),
            pl.BlockSpec((H, 256), lambda b: (0, 0)),
            pl.BlockSpec((1, 256), lambda b: (0, 0)),
            pl.BlockSpec((256, 1), lambda b: (0, 0)),
            pl.BlockSpec((1, 1), lambda b: (0, 0)),
            pl.BlockSpec((H, 4), lambda b: (0, 0)),
            pl.BlockSpec((1, 4), lambda b: (0, 0)),
        ],
        out_specs=[
            pl.BlockSpec((1, N2, 1), lambda b: (b, 0, 0)),
            pl.BlockSpec((1, 1, 1), lambda b: (b, 0, 0)),
            pl.BlockSpec((1, 1, 4), lambda b: (b, 0, 0)),
        ],
        out_shape=[
            jax.ShapeDtypeStruct((GB, N2, 1), f32),
            jax.ShapeDtypeStruct((GB, 1, 1), f32),
            jax.ShapeDtypeStruct((GB, 1, 4), f32),
        ],
    )
    hw = (params['Wq'], row2(params['bq']), params['Wp'], row2(params['bp']),
          params['W1'], row2(params['b1']), params['W2'], row2(params['b2']),
          params['Wt'], row2(params['bt']))
    pA, vA, tA = heads_call(hA, *hw)
    pB, vB, tB = heads_call(hB, *hw)

    p = jnp.concatenate([pA[..., 0], pB[..., 0]], axis=0)
    v = jnp.concatenate([vA[:, 0, :], vB[:, 0, :]], axis=0)
    t = jnp.concatenate([tA[:, 0, :], tB[:, 0, :]], axis=0)
    return (p, v, t)
